# baseline (speedup 1.0000x reference)
import functools

import jax
import jax.numpy as jnp
from jax import lax
from jax.experimental import pallas as pl
from jax.experimental.pallas import tpu as pltpu
from jax.experimental.pallas import tpu_sc as plsc

N_NODES = 10000
N_EDGES = 320000
D = 128

NC = 2
NS = 16
NW = NC * NS

EDGES_PER_TILE = N_EDGES // NW
CHUNK = 80
NSTEPS = EDGES_PER_TILE // CHUNK
NBUF = 3

RCHUNK = 80
NRCHUNKS = N_NODES // RCHUNK


def _sc_segment_sum(x, src, dst):
    mesh = plsc.VectorSubcoreMesh(core_axis_name="c", subcore_axis_name="s")

    @functools.partial(
        pl.kernel,
        mesh=mesh,
        out_type=jax.ShapeDtypeStruct((NC * N_NODES, D), jnp.float32),
        scratch_types=(
            [pltpu.VMEM((EDGES_PER_TILE,), jnp.int32)]
            + [pltpu.VMEM((CHUNK,), jnp.int32) for _ in range(NBUF)]
            + [pltpu.VMEM((CHUNK, D), jnp.float32) for _ in range(NBUF)]
            + [pltpu.VMEM_SHARED((N_NODES, D), jnp.float32)]
            + [pltpu.SemaphoreType.DMA for _ in range(3 * NBUF)]
        ),
    )
    def seg_sum(x_hbm, src_hbm, dst_hbm, out_hbm, src_all, *rest):
        dstbuf = rest[:NBUF]
        rows = rest[NBUF:2 * NBUF]
        acc = rest[2 * NBUF]
        gsem = rest[2 * NBUF + 1:2 * NBUF + 1 + NBUF]
        ssem = rest[2 * NBUF + 1 + NBUF:2 * NBUF + 1 + 2 * NBUF]
        isem = rest[2 * NBUF + 1 + 2 * NBUF:]
        rowsA = rows[0]
        c = lax.axis_index("c")
        s = lax.axis_index("s")
        base0 = c * (N_EDGES // NC) + s * EDGES_PER_TILE

        pltpu.sync_copy(src_hbm.at[pl.ds(base0, EDGES_PER_TILE)], src_all)

        zero = jnp.zeros((16,), jnp.float32)

        def zstep(i, carry):
            r = i // (D // 16)
            col = (i % (D // 16)) * 16
            rowsA[r, pl.ds(col, 16)] = zero
            return carry

        lax.fori_loop(0, CHUNK * (D // 16), zstep, 0)

        def zcopy(kk, carry):
            chunk = kk * NS + s
            @pl.when(chunk < NRCHUNKS)
            def _():
                pltpu.sync_copy(rowsA, acc.at[pl.ds(chunk * RCHUNK, RCHUNK)])
            return carry

        lax.fori_loop(0, (NRCHUNKS + NS - 1) // NS, zcopy, 0)
        plsc.subcore_barrier()

        def issue(j, b):
            pltpu.async_copy(
                dst_hbm.at[pl.ds(base0 + j * CHUNK, CHUNK)], dstbuf[b],
                isem[b])
            pltpu.async_copy(
                x_hbm.at[src_all.at[pl.ds(j * CHUNK, CHUNK)]], rows[b],
                gsem[b])

        def wait_in(b):
            pltpu.make_async_copy(
                dst_hbm.at[pl.ds(base0, CHUNK)], dstbuf[b], isem[b]).wait()
            pltpu.make_async_copy(
                x_hbm.at[src_all.at[pl.ds(0, CHUNK)]], rows[b],
                gsem[b]).wait()

        def scatter(b):
            pltpu.async_copy(rows[b], acc.at[dstbuf[b]], ssem[b], add=True)

        def wait_scatter(b):
            pltpu.make_async_copy(rows[b], acc.at[dstbuf[b]], ssem[b]).wait()

        for b in range(NBUF):
            issue(b, b)

        def body(i, carry):
            for b in range(NBUF):
                j = NBUF * i + b

                @pl.when(j < NSTEPS)
                def _(b=b):
                    wait_in(b)
                    scatter(b)

            for b in range(NBUF):
                j = NBUF * i + b

                @pl.when(j + NBUF < NSTEPS)
                def _(b=b, j=j):
                    wait_scatter(b)
                    issue(j + NBUF, b)

            return carry

        lax.fori_loop(0, (NSTEPS + NBUF - 1) // NBUF, body, 0)
        for b in range(NBUF):
            wait_scatter(b)
        plsc.subcore_barrier()

        def dcopy(kk, carry):
            chunk = kk * NS + s
            @pl.when(chunk < NRCHUNKS)
            def _():
                r0 = chunk * RCHUNK
                pltpu.sync_copy(
                    acc.at[pl.ds(r0, RCHUNK)],
                    out_hbm.at[pl.ds(c * N_NODES + r0, RCHUNK)])
            return carry

        lax.fori_loop(0, (NRCHUNKS + NS - 1) // NS, dcopy, 0)

    return seg_sum(x, src, dst)


def _xw1(x, W1t):
    BLK = 2000

    def body(x_ref, W1_ref, o_ref):
        o_ref[...] = jnp.dot(
            x_ref[...], W1_ref[...], preferred_element_type=jnp.float32)

    full = lambda i: (0, 0)
    return pl.pallas_call(
        body,
        grid=(N_NODES // BLK,),
        in_specs=[
            pl.BlockSpec((BLK, D), lambda i: (i, 0)),
            pl.BlockSpec((D, D), full),
        ],
        out_specs=pl.BlockSpec((BLK, D), lambda i: (i, 0)),
        out_shape=jax.ShapeDtypeStruct((N_NODES, D), jnp.float32),
    )(x, W1t)


def _mlp(eps, xw, parts, W1t, b1, gamma, beta, W2t, b2):
    BLK = 2000

    def body(eps_ref, xw_ref, p0_ref, p1_ref, W1_ref, b1_ref, g_ref, be_ref,
             W2_ref, b2_ref, o_ref):
        pw = jnp.dot(p0_ref[...] + p1_ref[...], W1_ref[...],
                     preferred_element_type=jnp.float32)
        h = xw_ref[...] * (1.0 + eps_ref[0]) + pw + b1_ref[...]
        mu = jnp.mean(h, axis=-1, keepdims=True)
        hc = h - mu
        var = jnp.mean(hc * hc, axis=-1, keepdims=True)
        h = hc * lax.rsqrt(var + 1e-5) * g_ref[...] + be_ref[...]
        h = jnp.maximum(h, 0.0)
        o_ref[...] = (
            jnp.dot(h, W2_ref[...], preferred_element_type=jnp.float32)
            + b2_ref[...])

    full = lambda i: (0, 0)
    nblk = N_NODES // BLK
    return pl.pallas_call(
        body,
        grid=(nblk,),
        in_specs=[
            pl.BlockSpec(memory_space=pltpu.SMEM),
            pl.BlockSpec((BLK, D), lambda i: (i, 0)),
            pl.BlockSpec((BLK, D), lambda i: (i, 0)),
            pl.BlockSpec((BLK, D), lambda i: (i + N_NODES // BLK, 0)),
            pl.BlockSpec((D, D), full),
            pl.BlockSpec((1, D), full),
            pl.BlockSpec((1, D), full),
            pl.BlockSpec((1, D), full),
            pl.BlockSpec((D, D), full),
            pl.BlockSpec((1, D), full),
        ],
        out_specs=pl.BlockSpec((BLK, D), lambda i: (i, 0)),
        out_shape=jax.ShapeDtypeStruct((N_NODES, D), jnp.float32),
    )(eps, xw, parts, parts, W1t, b1, gamma, beta, W2t, b2)


def kernel(x, edge_index, eps, W1, b1, gamma, beta, W2, b2):
    src = edge_index[0].astype(jnp.int32)
    dst = edge_index[1].astype(jnp.int32)
    W1t = W1.T
    xw = _xw1(x, W1t)
    parts = _sc_segment_sum(x, src, dst)
    return _mlp(
        eps.reshape(1), xw, parts,
        W1t, b1.reshape(1, D), gamma.reshape(1, D), beta.reshape(1, D),
        W2.T, b2.reshape(1, D))


# speedup vs baseline: 1.0535x; 1.0535x over previous
import functools

import jax
import jax.numpy as jnp
from jax import lax
from jax.experimental import pallas as pl
from jax.experimental.pallas import tpu as pltpu
from jax.experimental.pallas import tpu_sc as plsc

N_NODES = 10000
N_EDGES = 320000
D = 128

NC = 2
NS = 16
NW = NC * NS

EDGES_PER_TILE = N_EDGES // NW
CHUNK = 40
NSTEPS = EDGES_PER_TILE // CHUNK
NBUF = 6

RCHUNK = 40
NRCHUNKS = N_NODES // RCHUNK


def _sc_segment_sum(x, src, dst):
    mesh = plsc.VectorSubcoreMesh(core_axis_name="c", subcore_axis_name="s")

    @functools.partial(
        pl.kernel,
        mesh=mesh,
        out_type=jax.ShapeDtypeStruct((NC * N_NODES, D), jnp.float32),
        scratch_types=(
            [pltpu.VMEM((EDGES_PER_TILE,), jnp.int32)]
            + [pltpu.VMEM((CHUNK,), jnp.int32) for _ in range(NBUF)]
            + [pltpu.VMEM((CHUNK, D), jnp.float32) for _ in range(NBUF)]
            + [pltpu.VMEM_SHARED((N_NODES, D), jnp.float32)]
            + [pltpu.SemaphoreType.DMA for _ in range(3 * NBUF)]
        ),
    )
    def seg_sum(x_hbm, src_hbm, dst_hbm, out_hbm, src_all, *rest):
        dstbuf = rest[:NBUF]
        rows = rest[NBUF:2 * NBUF]
        acc = rest[2 * NBUF]
        gsem = rest[2 * NBUF + 1:2 * NBUF + 1 + NBUF]
        ssem = rest[2 * NBUF + 1 + NBUF:2 * NBUF + 1 + 2 * NBUF]
        isem = rest[2 * NBUF + 1 + 2 * NBUF:]
        c = lax.axis_index("c")
        s = lax.axis_index("s")
        base0 = c * (N_EDGES // NC) + s * EDGES_PER_TILE

        pltpu.sync_copy(src_hbm.at[pl.ds(base0, EDGES_PER_TILE)], src_all)

        def issue(j, b):
            pltpu.async_copy(
                dst_hbm.at[pl.ds(base0 + j * CHUNK, CHUNK)], dstbuf[b],
                isem[b])
            pltpu.async_copy(
                x_hbm.at[src_all.at[pl.ds(j * CHUNK, CHUNK)]], rows[b],
                gsem[b])

        def wait_in(b):
            pltpu.make_async_copy(
                dst_hbm.at[pl.ds(base0, CHUNK)], dstbuf[b], isem[b]).wait()
            pltpu.make_async_copy(
                x_hbm.at[src_all.at[pl.ds(0, CHUNK)]], rows[b],
                gsem[b]).wait()

        def scatter(b):
            pltpu.async_copy(rows[b], acc.at[dstbuf[b]], ssem[b], add=True)

        def wait_scatter(b):
            pltpu.make_async_copy(rows[b], acc.at[dstbuf[b]], ssem[b]).wait()

        for b in range(NBUF - 1):
            issue(b, b)

        zero = jnp.zeros((16,), jnp.float32)
        zrows = rows[NBUF - 1]

        def zstep(i, carry):
            r = i // (D // 16)
            col = (i % (D // 16)) * 16
            zrows[r, pl.ds(col, 16)] = zero
            return carry

        lax.fori_loop(0, CHUNK * (D // 16), zstep, 0)

        def zcopy(kk, carry):
            chunk = kk * NS + s
            @pl.when(chunk < NRCHUNKS)
            def _():
                pltpu.sync_copy(zrows, acc.at[pl.ds(chunk * RCHUNK, RCHUNK)])
            return carry

        lax.fori_loop(0, (NRCHUNKS + NS - 1) // NS, zcopy, 0)
        issue(NBUF - 1, NBUF - 1)
        plsc.subcore_barrier()

        def body(i, carry):
            for b in range(NBUF):
                j = NBUF * i + b

                @pl.when(j < NSTEPS)
                def _(b=b):
                    wait_in(b)
                    scatter(b)

            for b in range(NBUF):
                j = NBUF * i + b

                @pl.when(j + NBUF < NSTEPS)
                def _(b=b, j=j):
                    wait_scatter(b)
                    issue(j + NBUF, b)

            return carry

        lax.fori_loop(0, (NSTEPS + NBUF - 1) // NBUF, body, 0)
        for b in range(NBUF):
            wait_scatter(b)
        plsc.subcore_barrier()

        def dcopy(kk, carry):
            chunk = kk * NS + s
            @pl.when(chunk < NRCHUNKS)
            def _():
                r0 = chunk * RCHUNK
                pltpu.sync_copy(
                    acc.at[pl.ds(r0, RCHUNK)],
                    out_hbm.at[pl.ds(c * N_NODES + r0, RCHUNK)])
            return carry

        lax.fori_loop(0, (NRCHUNKS + NS - 1) // NS, dcopy, 0)

    return seg_sum(x, src, dst)


def _mlp(eps, x, parts, W1t, b1, gamma, beta, W2t, b2):
    BLK = 2000

    def body(eps_ref, x_ref, p0_ref, p1_ref, W1_ref, b1_ref, g_ref, be_ref,
             W2_ref, b2_ref, o_ref):
        h = x_ref[...] * (1.0 + eps_ref[0]) + p0_ref[...] + p1_ref[...]
        h = jnp.dot(h, W1_ref[...], preferred_element_type=jnp.float32)
        h = h + b1_ref[...]
        mu = jnp.mean(h, axis=-1, keepdims=True)
        hc = h - mu
        var = jnp.mean(hc * hc, axis=-1, keepdims=True)
        h = hc * lax.rsqrt(var + 1e-5) * g_ref[...] + be_ref[...]
        h = jnp.maximum(h, 0.0)
        o_ref[...] = (
            jnp.dot(h, W2_ref[...], preferred_element_type=jnp.float32)
            + b2_ref[...])

    full = lambda i: (0, 0)
    nblk = N_NODES // BLK
    return pl.pallas_call(
        body,
        grid=(nblk,),
        in_specs=[
            pl.BlockSpec(memory_space=pltpu.SMEM),
            pl.BlockSpec((BLK, D), lambda i: (i, 0)),
            pl.BlockSpec((BLK, D), lambda i: (i, 0)),
            pl.BlockSpec((BLK, D), lambda i: (i + N_NODES // BLK, 0)),
            pl.BlockSpec((D, D), full),
            pl.BlockSpec((1, D), full),
            pl.BlockSpec((1, D), full),
            pl.BlockSpec((1, D), full),
            pl.BlockSpec((D, D), full),
            pl.BlockSpec((1, D), full),
        ],
        out_specs=pl.BlockSpec((BLK, D), lambda i: (i, 0)),
        out_shape=jax.ShapeDtypeStruct((N_NODES, D), jnp.float32),
    )(eps, x, parts, parts, W1t, b1, gamma, beta, W2t, b2)


def kernel(x, edge_index, eps, W1, b1, gamma, beta, W2, b2):
    src = edge_index[0].astype(jnp.int32)
    dst = edge_index[1].astype(jnp.int32)
    parts = _sc_segment_sum(x, src, dst)
    return _mlp(
        eps.reshape(1), x, parts,
        W1.T, b1.reshape(1, D), gamma.reshape(1, D), beta.reshape(1, D),
        W2.T, b2.reshape(1, D))


# speedup vs baseline: 1.0647x; 1.0106x over previous
import functools

import jax
import jax.numpy as jnp
from jax import lax
from jax.experimental import pallas as pl
from jax.experimental.pallas import tpu as pltpu
from jax.experimental.pallas import tpu_sc as plsc

N_NODES = 10000
N_EDGES = 320000
D = 128

NC = 2
NS = 16
NW = NC * NS

EDGES_PER_TILE = N_EDGES // NW
CHUNK = 40
NSTEPS = EDGES_PER_TILE // CHUNK
NBUF = 7

RCHUNK = 40
NRCHUNKS = N_NODES // RCHUNK


def _sc_segment_sum(x, src, dst):
    mesh = plsc.VectorSubcoreMesh(core_axis_name="c", subcore_axis_name="s")

    @functools.partial(
        pl.kernel,
        mesh=mesh,
        out_type=jax.ShapeDtypeStruct((NC * N_NODES, D), jnp.float32),
        scratch_types=(
            [pltpu.VMEM((EDGES_PER_TILE,), jnp.int32)]
            + [pltpu.VMEM((CHUNK,), jnp.int32) for _ in range(NBUF)]
            + [pltpu.VMEM((CHUNK, D), jnp.float32) for _ in range(NBUF)]
            + [pltpu.VMEM_SHARED((N_NODES, D), jnp.float32)]
            + [pltpu.SemaphoreType.DMA for _ in range(3 * NBUF)]
        ),
    )
    def seg_sum(x_hbm, src_hbm, dst_hbm, out_hbm, src_all, *rest):
        dstbuf = rest[:NBUF]
        rows = rest[NBUF:2 * NBUF]
        acc = rest[2 * NBUF]
        gsem = rest[2 * NBUF + 1:2 * NBUF + 1 + NBUF]
        ssem = rest[2 * NBUF + 1 + NBUF:2 * NBUF + 1 + 2 * NBUF]
        isem = rest[2 * NBUF + 1 + 2 * NBUF:]
        c = lax.axis_index("c")
        s = lax.axis_index("s")
        base0 = c * (N_EDGES // NC) + s * EDGES_PER_TILE

        pltpu.sync_copy(src_hbm.at[pl.ds(base0, EDGES_PER_TILE)], src_all)

        def issue(j, b):
            pltpu.async_copy(
                dst_hbm.at[pl.ds(base0 + j * CHUNK, CHUNK)], dstbuf[b],
                isem[b])
            pltpu.async_copy(
                x_hbm.at[src_all.at[pl.ds(j * CHUNK, CHUNK)]], rows[b],
                gsem[b])

        def wait_in(b):
            pltpu.make_async_copy(
                dst_hbm.at[pl.ds(base0, CHUNK)], dstbuf[b], isem[b]).wait()
            pltpu.make_async_copy(
                x_hbm.at[src_all.at[pl.ds(0, CHUNK)]], rows[b],
                gsem[b]).wait()

        def scatter(b):
            pltpu.async_copy(rows[b], acc.at[dstbuf[b]], ssem[b], add=True)

        def wait_scatter(b):
            pltpu.make_async_copy(rows[b], acc.at[dstbuf[b]], ssem[b]).wait()

        for b in range(NBUF - 1):
            issue(b, b)

        zero = jnp.zeros((16,), jnp.float32)
        zrows = rows[NBUF - 1]

        def zstep(i, carry):
            r = i // (D // 16)
            col = (i % (D // 16)) * 16
            zrows[r, pl.ds(col, 16)] = zero
            return carry

        lax.fori_loop(0, CHUNK * (D // 16), zstep, 0)

        def zcopy(kk, carry):
            chunk = kk * NS + s
            @pl.when(chunk < NRCHUNKS)
            def _():
                pltpu.sync_copy(zrows, acc.at[pl.ds(chunk * RCHUNK, RCHUNK)])
            return carry

        lax.fori_loop(0, (NRCHUNKS + NS - 1) // NS, zcopy, 0)
        issue(NBUF - 1, NBUF - 1)
        plsc.subcore_barrier()

        def body(i, carry):
            for b in range(NBUF):
                j = NBUF * i + b

                @pl.when(j < NSTEPS)
                def _(b=b):
                    wait_in(b)
                    scatter(b)

            for b in range(NBUF):
                j = NBUF * i + b

                @pl.when(j + NBUF < NSTEPS)
                def _(b=b, j=j):
                    wait_scatter(b)
                    issue(j + NBUF, b)

            return carry

        lax.fori_loop(0, (NSTEPS + NBUF - 1) // NBUF, body, 0)
        for b in range(NBUF):
            wait_scatter(b)
        plsc.subcore_barrier()

        def dcopy(kk, carry):
            chunk = kk * NS + s
            @pl.when(chunk < NRCHUNKS)
            def _():
                r0 = chunk * RCHUNK
                pltpu.sync_copy(
                    acc.at[pl.ds(r0, RCHUNK)],
                    out_hbm.at[pl.ds(c * N_NODES + r0, RCHUNK)])
            return carry

        lax.fori_loop(0, (NRCHUNKS + NS - 1) // NS, dcopy, 0)

    return seg_sum(x, src, dst)


def _mlp(eps, x, parts, W1t, b1, gamma, beta, W2t, b2):
    BLK = 2000

    def body(eps_ref, x_ref, p0_ref, p1_ref, W1_ref, b1_ref, g_ref, be_ref,
             W2_ref, b2_ref, o_ref):
        h = x_ref[...] * (1.0 + eps_ref[0]) + p0_ref[...] + p1_ref[...]
        h = jnp.dot(h, W1_ref[...], preferred_element_type=jnp.float32)
        h = h + b1_ref[...]
        mu = jnp.mean(h, axis=-1, keepdims=True)
        hc = h - mu
        var = jnp.mean(hc * hc, axis=-1, keepdims=True)
        h = hc * lax.rsqrt(var + 1e-5) * g_ref[...] + be_ref[...]
        h = jnp.maximum(h, 0.0)
        o_ref[...] = (
            jnp.dot(h, W2_ref[...], preferred_element_type=jnp.float32)
            + b2_ref[...])

    full = lambda i: (0, 0)
    nblk = N_NODES // BLK
    return pl.pallas_call(
        body,
        grid=(nblk,),
        in_specs=[
            pl.BlockSpec(memory_space=pltpu.SMEM),
            pl.BlockSpec((BLK, D), lambda i: (i, 0)),
            pl.BlockSpec((BLK, D), lambda i: (i, 0)),
            pl.BlockSpec((BLK, D), lambda i: (i + N_NODES // BLK, 0)),
            pl.BlockSpec((D, D), full),
            pl.BlockSpec((1, D), full),
            pl.BlockSpec((1, D), full),
            pl.BlockSpec((1, D), full),
            pl.BlockSpec((D, D), full),
            pl.BlockSpec((1, D), full),
        ],
        out_specs=pl.BlockSpec((BLK, D), lambda i: (i, 0)),
        out_shape=jax.ShapeDtypeStruct((N_NODES, D), jnp.float32),
    )(eps, x, parts, parts, W1t, b1, gamma, beta, W2t, b2)


def kernel(x, edge_index, eps, W1, b1, gamma, beta, W2, b2):
    src = edge_index[0].astype(jnp.int32)
    dst = edge_index[1].astype(jnp.int32)
    parts = _sc_segment_sum(x, src, dst)
    return _mlp(
        eps.reshape(1), x, parts,
        W1.T, b1.reshape(1, D), gamma.reshape(1, D), beta.reshape(1, D),
        W2.T, b2.reshape(1, D))


# speedup vs baseline: 1.0733x; 1.0080x over previous
import functools

import jax
import jax.numpy as jnp
from jax import lax
from jax.experimental import pallas as pl
from jax.experimental.pallas import tpu as pltpu
from jax.experimental.pallas import tpu_sc as plsc

N_NODES = 10000
N_EDGES = 320000
D = 128

NC = 2
NS = 16
NW = NC * NS

EDGES_PER_TILE = N_EDGES // NW
CHUNK = 40
NSTEPS = EDGES_PER_TILE // CHUNK
NBUF = 7

RCHUNK = 40
NRCHUNKS = N_NODES // RCHUNK
DCHUNK = 400
NDCHUNKS = N_NODES // DCHUNK


def _sc_segment_sum(x, src, dst):
    mesh = plsc.VectorSubcoreMesh(core_axis_name="c", subcore_axis_name="s")

    @functools.partial(
        pl.kernel,
        mesh=mesh,
        out_type=jax.ShapeDtypeStruct((NC * N_NODES, D), jnp.float32),
        scratch_types=(
            [pltpu.VMEM((EDGES_PER_TILE,), jnp.int32)]
            + [pltpu.VMEM((CHUNK,), jnp.int32) for _ in range(NBUF)]
            + [pltpu.VMEM((CHUNK, D), jnp.float32) for _ in range(NBUF)]
            + [pltpu.VMEM_SHARED((N_NODES, D), jnp.float32)]
            + [pltpu.SemaphoreType.DMA for _ in range(3 * NBUF)]
        ),
    )
    def seg_sum(x_hbm, src_hbm, dst_hbm, out_hbm, src_all, *rest):
        dstbuf = rest[:NBUF]
        rows = rest[NBUF:2 * NBUF]
        acc = rest[2 * NBUF]
        gsem = rest[2 * NBUF + 1:2 * NBUF + 1 + NBUF]
        ssem = rest[2 * NBUF + 1 + NBUF:2 * NBUF + 1 + 2 * NBUF]
        isem = rest[2 * NBUF + 1 + 2 * NBUF:]
        c = lax.axis_index("c")
        s = lax.axis_index("s")
        base0 = c * (N_EDGES // NC) + s * EDGES_PER_TILE

        pltpu.sync_copy(src_hbm.at[pl.ds(base0, EDGES_PER_TILE)], src_all)

        def issue(j, b):
            pltpu.async_copy(
                dst_hbm.at[pl.ds(base0 + j * CHUNK, CHUNK)], dstbuf[b],
                isem[b])
            pltpu.async_copy(
                x_hbm.at[src_all.at[pl.ds(j * CHUNK, CHUNK)]], rows[b],
                gsem[b])

        def wait_in(b):
            pltpu.make_async_copy(
                dst_hbm.at[pl.ds(base0, CHUNK)], dstbuf[b], isem[b]).wait()
            pltpu.make_async_copy(
                x_hbm.at[src_all.at[pl.ds(0, CHUNK)]], rows[b],
                gsem[b]).wait()

        def scatter(b):
            pltpu.async_copy(rows[b], acc.at[dstbuf[b]], ssem[b], add=True)

        def wait_scatter(b):
            pltpu.make_async_copy(rows[b], acc.at[dstbuf[b]], ssem[b]).wait()

        for b in range(NBUF - 1):
            issue(b, b)

        zero = jnp.zeros((16,), jnp.float32)
        zrows = rows[NBUF - 1]

        def zstep(i, carry):
            r = i // (D // 16)
            col = (i % (D // 16)) * 16
            zrows[r, pl.ds(col, 16)] = zero
            return carry

        lax.fori_loop(0, CHUNK * (D // 16), zstep, 0)

        def zcopy(kk, carry):
            chunk = kk * NS + s
            @pl.when(chunk < NRCHUNKS)
            def _():
                pltpu.sync_copy(zrows, acc.at[pl.ds(chunk * RCHUNK, RCHUNK)])
            return carry

        lax.fori_loop(0, (NRCHUNKS + NS - 1) // NS, zcopy, 0)
        issue(NBUF - 1, NBUF - 1)
        plsc.subcore_barrier()

        def body(i, carry):
            for b in range(NBUF):
                j = NBUF * i + b

                @pl.when(j < NSTEPS)
                def _(b=b):
                    wait_in(b)
                    scatter(b)

            for b in range(NBUF):
                j = NBUF * i + b

                @pl.when(j + NBUF < NSTEPS)
                def _(b=b, j=j):
                    wait_scatter(b)
                    issue(j + NBUF, b)

            return carry

        lax.fori_loop(0, (NSTEPS + NBUF - 1) // NBUF, body, 0)
        for b in range(NBUF):
            wait_scatter(b)
        plsc.subcore_barrier()

        def dcopy(kk, carry):
            chunk = kk * NS + s
            @pl.when(chunk < NDCHUNKS)
            def _():
                r0 = chunk * DCHUNK
                pltpu.sync_copy(
                    acc.at[pl.ds(r0, DCHUNK)],
                    out_hbm.at[pl.ds(c * N_NODES + r0, DCHUNK)])
            return carry

        lax.fori_loop(0, (NDCHUNKS + NS - 1) // NS, dcopy, 0)

    return seg_sum(x, src, dst)


def _mlp(eps, x, parts, W1t, b1, gamma, beta, W2t, b2):
    BLK = 1000

    def body(eps_ref, x_ref, p0_ref, p1_ref, W1_ref, b1_ref, g_ref, be_ref,
             W2_ref, b2_ref, o_ref):
        h = x_ref[...] * (1.0 + eps_ref[0]) + p0_ref[...] + p1_ref[...]
        h = jnp.dot(h, W1_ref[...], preferred_element_type=jnp.float32)
        h = h + b1_ref[...]
        mu = jnp.mean(h, axis=-1, keepdims=True)
        hc = h - mu
        var = jnp.mean(hc * hc, axis=-1, keepdims=True)
        h = hc * lax.rsqrt(var + 1e-5) * g_ref[...] + be_ref[...]
        h = jnp.maximum(h, 0.0)
        o_ref[...] = (
            jnp.dot(h, W2_ref[...], preferred_element_type=jnp.float32)
            + b2_ref[...])

    full = lambda i: (0, 0)
    nblk = N_NODES // BLK
    return pl.pallas_call(
        body,
        grid=(nblk,),
        in_specs=[
            pl.BlockSpec(memory_space=pltpu.SMEM),
            pl.BlockSpec((BLK, D), lambda i: (i, 0)),
            pl.BlockSpec((BLK, D), lambda i: (i, 0)),
            pl.BlockSpec((BLK, D), lambda i: (i + N_NODES // BLK, 0)),
            pl.BlockSpec((D, D), full),
            pl.BlockSpec((1, D), full),
            pl.BlockSpec((1, D), full),
            pl.BlockSpec((1, D), full),
            pl.BlockSpec((D, D), full),
            pl.BlockSpec((1, D), full),
        ],
        out_specs=pl.BlockSpec((BLK, D), lambda i: (i, 0)),
        out_shape=jax.ShapeDtypeStruct((N_NODES, D), jnp.float32),
    )(eps, x, parts, parts, W1t, b1, gamma, beta, W2t, b2)


def kernel(x, edge_index, eps, W1, b1, gamma, beta, W2, b2):
    src = edge_index[0].astype(jnp.int32)
    dst = edge_index[1].astype(jnp.int32)
    parts = _sc_segment_sum(x, src, dst)
    return _mlp(
        eps.reshape(1), x, parts,
        W1.T, b1.reshape(1, D), gamma.reshape(1, D), beta.reshape(1, D),
        W2.T, b2.reshape(1, D))
